# Initial kernel scaffold; baseline (speedup 1.0000x reference)
import jax
import jax.numpy as jnp
from jax.experimental import pallas as pl


def kernel(x1, x2, x3):
    raise NotImplementedError("write your pallas kernel here")


# speedup vs baseline: 2.3321x; 2.3321x over previous
import jax
import jax.numpy as jnp
import numpy as np
from jax.experimental import pallas as pl
from jax.experimental.pallas import tpu as pltpu

_INP_DIM = 608
_NCH = 85
_B = 16
_SCALES = (
    (76, ((10.0, 13.0), (16.0, 30.0), (33.0, 23.0))),
    (38, ((30.0, 61.0), (62.0, 45.0), (59.0, 119.0))),
    (19, ((116.0, 90.0), (156.0, 198.0), (373.0, 326.0))),
)


def _grid_table(W: int) -> np.ndarray:
    hw = W * W
    s = np.arange(hw)
    g = np.zeros((8, hw), np.float32)
    g[0] = s % W
    g[1] = s // W
    return g


def _sigmoid(x):
    return 1.0 / (1.0 + jnp.exp(-x))


def _decode_plane(v, g, W, aw, ah):
    stride = float(_INP_DIM // W)
    h = v[:8]
    sigh = _sigmoid(h)
    exh = jnp.exp(h)
    row8 = jax.lax.broadcasted_iota(jnp.int32, h.shape, 0)
    hout = jnp.where(
        row8 < 2, (sigh + g) * stride,
        jnp.where(row8 == 2, exh * aw,
                  jnp.where(row8 == 3, exh * ah, sigh)))
    return jnp.concatenate([hout, _sigmoid(v[8:])], axis=0)


def _body(x1_ref, x2_ref, x3_ref, g1_ref, g2_ref, g3_ref,
          d1_ref, d2_ref, d3_ref):
    a = pl.program_id(0) % 3
    for x_ref, g_ref, d_ref, (W, anc) in (
            (x1_ref, g1_ref, d1_ref, _SCALES[0]),
            (x2_ref, g2_ref, d2_ref, _SCALES[1]),
            (x3_ref, g3_ref, d3_ref, _SCALES[2])):
        aw = jnp.where(a == 0, anc[0][0], jnp.where(a == 1, anc[1][0], anc[2][0]))
        ah = jnp.where(a == 0, anc[0][1], jnp.where(a == 1, anc[1][1], anc[2][1]))
        p = _decode_plane(x_ref[0, 0], g_ref[...], W, aw, ah)
        d_ref[0, 0] = p.T


def _iomap(i):
    return (i // 3, i % 3, 0, 0)


@jax.jit
def kernel(x1, x2, x3):
    B = x1.shape[0]
    hws = tuple(W * W for W, _ in _SCALES)
    xs = [x.reshape(B, 3, _NCH, hw) for x, hw in zip((x1, x2, x3), hws)]
    tables = [jnp.asarray(_grid_table(W)) for W, _ in _SCALES]

    d1, d2, d3 = pl.pallas_call(
        _body,
        grid=(B * 3,),
        in_specs=[
            *[pl.BlockSpec((1, 1, _NCH, hw), _iomap) for hw in hws],
            *[pl.BlockSpec((8, hw), lambda i: (0, 0)) for hw in hws],
        ],
        out_specs=[
            pl.BlockSpec((1, 1, hw, _NCH), _iomap) for hw in hws
        ],
        out_shape=[
            jax.ShapeDtypeStruct((B, 3, hw, _NCH), jnp.float32) for hw in hws
        ],
        compiler_params=pltpu.CompilerParams(
            dimension_semantics=("arbitrary",),
        ),
        name="yolo_decode",
    )(*xs, *tables)

    return jnp.concatenate(
        [d.reshape(B, -1, _NCH) for d in (d1, d2, d3)], axis=1)


# speedup vs baseline: 3.2668x; 1.4008x over previous
import jax
import jax.numpy as jnp
import numpy as np
from jax.experimental import pallas as pl
from jax.experimental.pallas import tpu as pltpu

_INP_DIM = 608
_NCH = 85
_ROWS = 22743
_ANCHORS = {
    76: ((10.0, 13.0), (16.0, 30.0), (33.0, 23.0)),
    38: ((30.0, 61.0), (62.0, 45.0), (59.0, 119.0)),
    19: ((116.0, 90.0), (156.0, 198.0), (373.0, 326.0)),
}
_WS = (76, 38, 19)


def _grid_tables(W):
    s = np.arange(W * W)
    gx = np.tile((s % W)[None, :].astype(np.float32), (8, 1))
    gy = np.tile((s // W)[None, :].astype(np.float32), (8, 1))
    return gx, gy


def _sigmoid(x):
    return 1.0 / (1.0 + jnp.exp(-x))


def _body(x1_ref, x2_ref, x3_ref, gx1_ref, gy1_ref, gx2_ref, gy2_ref,
          gx3_ref, gy3_ref, o_ref):
    k = pl.program_id(0)
    scales = (
        (x1_ref, gx1_ref, gy1_ref, 76),
        (x2_ref, gx2_ref, gy2_ref, 38),
        (x3_ref, gx3_ref, gy3_ref, 19),
    )

    def emit(is_head):
        pieces = []
        for x_ref, gx_ref, gy_ref, W in scales:
            stride = float(_INP_DIM // W)
            for a in range(3):
                va = x_ref[a]
                if is_head:
                    aw, ah = _ANCHORS[W][a]
                    bx = (_sigmoid(va[0]) + gx_ref[...]) * stride
                    by = (_sigmoid(va[1]) + gy_ref[...]) * stride
                    bw = jnp.exp(va[2]) * aw
                    bh = jnp.exp(va[3]) * ah
                    pieces.append(jnp.concatenate(
                        [bx[None], by[None], bw[None], bh[None],
                         _sigmoid(va[4:])], axis=0))
                else:
                    pieces.append(_sigmoid(va))
        o_ref[...] = jnp.concatenate(pieces, axis=2)

    @pl.when(k == 0)
    def _():
        emit(True)

    @pl.when(k != 0)
    def _():
        emit(False)


@jax.jit
def kernel(x1, x2, x3):
    B = x1.shape[0]
    xs = [
        jnp.transpose(x.reshape(B, 3, _NCH, W * W), (1, 2, 0, 3))
        for x, W in zip((x1, x2, x3), _WS)
    ]
    tables = [jnp.asarray(t) for W in _WS for t in _grid_tables(W)]

    res = pl.pallas_call(
        _body,
        grid=(11, B // 8),
        in_specs=[
            *[pl.BlockSpec((3, 8, 8, W * W), lambda k, b: (0, k, b, 0))
              for W in _WS],
            *[pl.BlockSpec((8, W * W), lambda k, b: (0, 0))
              for W in _WS for _ in range(2)],
        ],
        out_specs=pl.BlockSpec((8, 8, _ROWS), lambda k, b: (k, b, 0)),
        out_shape=jax.ShapeDtypeStruct((_NCH, B, _ROWS), jnp.float32),
        compiler_params=pltpu.CompilerParams(
            dimension_semantics=("parallel", "parallel"),
            vmem_limit_bytes=50 * 1024 * 1024,
        ),
        name="yolo_decode",
    )(*xs, *tables)

    return jnp.transpose(res, (1, 2, 0))


# speedup vs baseline: 3.3120x; 1.0138x over previous
import jax
import jax.numpy as jnp
import numpy as np
from jax.experimental import pallas as pl
from jax.experimental.pallas import tpu as pltpu

_INP_DIM = 608
_NCH = 85
_ROWS = 22743
_ANCHORS = {
    76: ((10.0, 13.0), (16.0, 30.0), (33.0, 23.0)),
    38: ((30.0, 61.0), (62.0, 45.0), (59.0, 119.0)),
    19: ((116.0, 90.0), (156.0, 198.0), (373.0, 326.0)),
}
_WS = (76, 38, 19)


def _full_tables():
    gx, gy, ts, aw, ah = [], [], [], [], []
    for W in _WS:
        s = np.arange(W * W)
        stride = float(_INP_DIM // W)
        for a in range(3):
            gx.append(s % W)
            gy.append(s // W)
            ts.append(np.full(W * W, stride))
            aw.append(np.full(W * W, _ANCHORS[W][a][0]))
            ah.append(np.full(W * W, _ANCHORS[W][a][1]))
    return [
        np.tile(np.concatenate(v)[None, :].astype(np.float32), (8, 1))
        for v in (gx, gy, ts, aw, ah)
    ]


def _sigmoid(x):
    return 1.0 / (1.0 + jnp.exp(-x))


def _body(x1_ref, x2_ref, x3_ref, gx_ref, gy_ref, ts_ref, aw_ref, ah_ref,
          o_ref, b1, b2, b3, sem):
    k = pl.program_id(0)
    bh = pl.program_id(1)
    srcs = ((x1_ref, b1), (x2_ref, b2), (x3_ref, b3))

    def dma_list():
        row0 = pl.multiple_of(k * 8, 8)
        copies = []
        for x_hbm, buf in srcs:
            for a in range(3):
                for b in range(8):
                    copies.append(pltpu.make_async_copy(
                        x_hbm.at[bh * 8 + b, a, pl.ds(row0, 8), :],
                        buf.at[a, :, b, :],
                        sem))
        return copies

    for c in dma_list():
        c.start()
    for c in dma_list():
        c.wait()

    full = jnp.concatenate(
        [buf[a] for _, buf in srcs for a in range(3)], axis=2)
    sig = _sigmoid(full)

    @pl.when(k == 0)
    def _():
        bx = (sig[0] + gx_ref[...]) * ts_ref[...]
        by = (sig[1] + gy_ref[...]) * ts_ref[...]
        bw = jnp.exp(full[2]) * aw_ref[...]
        bh_ = jnp.exp(full[3]) * ah_ref[...]
        o_ref[...] = jnp.concatenate(
            [bx[None], by[None], bw[None], bh_[None], sig[4:]], axis=0)

    @pl.when(k != 0)
    def _():
        o_ref[...] = sig


@jax.jit
def kernel(x1, x2, x3):
    B = x1.shape[0]
    xs = [x.reshape(B, 3, _NCH, W * W) for x, W in zip((x1, x2, x3), _WS)]
    tables = [jnp.asarray(t) for t in _full_tables()]

    res = pl.pallas_call(
        _body,
        grid=(11, B // 8),
        in_specs=[
            *[pl.BlockSpec(memory_space=pl.ANY) for _ in range(3)],
            *[pl.BlockSpec((8, _ROWS), lambda k, b: (0, 0))
              for _ in range(5)],
        ],
        out_specs=pl.BlockSpec((8, 8, _ROWS), lambda k, b: (k, b, 0)),
        out_shape=jax.ShapeDtypeStruct((_NCH, B, _ROWS), jnp.float32),
        scratch_shapes=[
            pltpu.VMEM((3, 8, 8, W * W), jnp.float32) for W in _WS
        ] + [pltpu.SemaphoreType.DMA],
        compiler_params=pltpu.CompilerParams(
            dimension_semantics=("arbitrary", "arbitrary"),
            vmem_limit_bytes=50 * 1024 * 1024,
        ),
        name="yolo_decode",
    )(*xs, *tables)

    return jnp.transpose(res, (1, 2, 0))


# speedup vs baseline: 3.9849x; 1.2032x over previous
import jax
import jax.numpy as jnp
from jax.experimental import pallas as pl
from jax.experimental.pallas import tpu as pltpu

_INP_DIM = 608
_NCH = 85
_ROWS = 22743
_BLK = 5776
_ANCH_S = ((10.0, 13.0), (16.0, 30.0), (33.0, 23.0))
_ANCH_M = ((30.0, 61.0), (62.0, 45.0), (59.0, 119.0))
_ANCH_L = ((116.0, 90.0), (156.0, 198.0), (373.0, 326.0))


def _sigmoid(x):
    return 1.0 / (1.0 + jnp.exp(-x))


def _decode_plane(v, W, aw, ah):
    stride = float(_INP_DIM // W)
    h = v[:8]
    sigh = _sigmoid(h)
    exh = jnp.exp(h)
    gx = jax.lax.broadcasted_iota(jnp.int32, h.shape, 2).astype(jnp.float32)
    gy = jax.lax.broadcasted_iota(jnp.int32, h.shape, 1).astype(jnp.float32)
    row8 = jax.lax.broadcasted_iota(jnp.int32, h.shape, 0)
    hout = jnp.where(
        row8 == 0, (sigh + gx) * stride,
        jnp.where(row8 == 1, (sigh + gy) * stride,
                  jnp.where(row8 == 2, exh * aw,
                            jnp.where(row8 == 3, exh * ah, sigh))))
    p = jnp.concatenate([hout, _sigmoid(v[8:])], axis=0)
    return jnp.transpose(p, (1, 2, 0)).reshape(W * W, _NCH)


def _body_s(x1_ref, o_ref):
    a = pl.program_id(1)
    aw = jnp.where(a == 0, _ANCH_S[0][0],
                   jnp.where(a == 1, _ANCH_S[1][0], _ANCH_S[2][0]))
    ah = jnp.where(a == 0, _ANCH_S[0][1],
                   jnp.where(a == 1, _ANCH_S[1][1], _ANCH_S[2][1]))
    o_ref[0] = _decode_plane(x1_ref[0], 76, aw, ah)


def _body_ml(x2_ref, x3_ref, buf_ref, o_ref):
    del buf_ref
    for a in range(3):
        o_ref[0, 1444 * a:1444 * (a + 1)] = _decode_plane(
            x2_ref[0, 85 * a:85 * (a + 1)], 38, *_ANCH_M[a])
    for a in range(3):
        o_ref[0, 4332 + 361 * a:4332 + 361 * (a + 1)] = _decode_plane(
            x3_ref[0, 85 * a:85 * (a + 1)], 19, *_ANCH_L[a])


@jax.jit
def kernel(x1, x2, x3):
    B = x1.shape[0]
    out_sds = jax.ShapeDtypeStruct((B, _ROWS, _NCH), jnp.float32)

    out = pl.pallas_call(
        _body_s,
        grid=(B, 3),
        in_specs=[pl.BlockSpec((1, _NCH, 76, 76), lambda b, a: (b, a, 0, 0))],
        out_specs=pl.BlockSpec((1, _BLK, _NCH), lambda b, a: (b, a, 0)),
        out_shape=out_sds,
        compiler_params=pltpu.CompilerParams(
            dimension_semantics=("parallel", "arbitrary")),
        name="yolo_decode_s",
    )(x1)

    out = pl.pallas_call(
        _body_ml,
        grid=(B,),
        in_specs=[
            pl.BlockSpec((1, 3 * _NCH, 38, 38), lambda b: (b, 0, 0, 0)),
            pl.BlockSpec((1, 3 * _NCH, 19, 19), lambda b: (b, 0, 0, 0)),
            pl.BlockSpec(memory_space=pl.ANY),
        ],
        out_specs=pl.BlockSpec((1, _BLK, _NCH), lambda b: (b, 3, 0)),
        out_shape=out_sds,
        input_output_aliases={2: 0},
        compiler_params=pltpu.CompilerParams(
            dimension_semantics=("parallel",)),
        name="yolo_decode_ml",
    )(x2, x3, out)

    return out


# speedup vs baseline: 5.7580x; 1.4450x over previous
import jax
import jax.numpy as jnp
import numpy as np
from jax.experimental import pallas as pl
from jax.experimental.pallas import tpu as pltpu

_INP_DIM = 608
_NCH = 85
_ROWS = 22743
_ANCHORS = {
    76: ((10.0, 13.0), (16.0, 30.0), (33.0, 23.0)),
    38: ((30.0, 61.0), (62.0, 45.0), (59.0, 119.0)),
    19: ((116.0, 90.0), (156.0, 198.0), (373.0, 326.0)),
}
_WS = (76, 38, 19)


def _grid_table(W):
    hw = W * W
    s = np.arange(hw)
    g = np.zeros((8, hw), np.float32)
    g[0] = s % W
    g[1] = s // W
    return g


def _sigmoid(x):
    return 1.0 / (1.0 + jnp.exp(-x))


def _decode_plane(v, g, W, aw, ah):
    stride = float(_INP_DIM // W)
    h = v[:8]
    sigh = _sigmoid(h)
    exh = jnp.exp(h)
    row8 = jax.lax.broadcasted_iota(jnp.int32, h.shape, 0)
    hout = jnp.where(
        row8 == 0, (sigh + g[0:1]) * stride,
        jnp.where(row8 == 1, (sigh + g[1:2]) * stride,
                  jnp.where(row8 == 2, exh * aw,
                            jnp.where(row8 == 3, exh * ah, sigh))))
    p = jnp.concatenate([hout, _sigmoid(v[8:])], axis=0)
    return jnp.transpose(p)


def _body(x1_ref, x2_ref, x3_ref, g1_ref, g2_ref, g3_ref, o_ref):
    row = 0
    for x_ref, g_ref, W in ((x1_ref, g1_ref, 76), (x2_ref, g2_ref, 38),
                            (x3_ref, g3_ref, 19)):
        hw = W * W
        for a in range(3):
            aw, ah = _ANCHORS[W][a]
            o_ref[0, row:row + hw] = _decode_plane(
                x_ref[0, 85 * a:85 * (a + 1)], g_ref[...], W, aw, ah)
            row += hw


@jax.jit
def kernel(x1, x2, x3):
    B = x1.shape[0]
    xs = [x.reshape(B, 3 * _NCH, W * W) for x, W in zip((x1, x2, x3), _WS)]
    tables = [jnp.asarray(_grid_table(W)) for W in _WS]

    res = pl.pallas_call(
        _body,
        grid=(B,),
        in_specs=[
            *[pl.BlockSpec((1, 3 * _NCH, W * W), lambda b: (b, 0, 0))
              for W in _WS],
            *[pl.BlockSpec((8, W * W), lambda b: (0, 0)) for W in _WS],
        ],
        out_specs=pl.BlockSpec((1, _ROWS, _NCH), lambda b: (b, 0, 0)),
        out_shape=jax.ShapeDtypeStruct((B, _ROWS, _NCH), jnp.float32),
        compiler_params=pltpu.CompilerParams(
            dimension_semantics=("parallel",),
            vmem_limit_bytes=56 * 1024 * 1024,
        ),
        name="yolo_decode",
    )(*xs, *tables)

    return res
